# baseline (speedup 1.0000x reference)
import functools

import jax
import jax.numpy as jnp
from jax import lax
from jax.experimental import pallas as pl
from jax.experimental.pallas import tpu as pltpu
from jax.experimental.pallas import tpu_sc as plsc

NC, NS, L = 2, 16, 16
NW = NC * NS
D = 1024
ROWS = 4 * 2048
RPW = ROWS // NW
CH = 16
NCH = RPW // CH
VPR = D // L


def _sc_body(x_hbm, idx_hbm, pe_hbm, out_hbm, idx_v, pe_v, x_v, gsem):
    wid = lax.axis_index("s") * NC + lax.axis_index("c")
    base = wid * RPW
    pltpu.sync_copy(idx_hbm.at[pl.ds(base, RPW)], idx_v)

    def chunk(c, carry):
        row0 = base + c * CH
        gcp = pltpu.async_copy(pe_hbm.at[idx_v.at[pl.ds(c * CH, CH)]], pe_v, gsem)
        pltpu.sync_copy(x_hbm.at[pl.ds(row0, CH)], x_v)
        gcp.wait()

        def addrow(r, carry_r):
            def addvec(j, carry_j):
                x_v[r, pl.ds(j * L, L)] = (
                    x_v[r, pl.ds(j * L, L)] + pe_v[r, pl.ds(j * L, L)]
                )
                return carry_j

            return lax.fori_loop(0, VPR, addvec, carry_r)

        lax.fori_loop(0, CH, addrow, 0)
        pltpu.sync_copy(x_v, out_hbm.at[pl.ds(row0, CH)])
        return carry

    lax.fori_loop(0, NCH, chunk, 0)


@functools.partial(jax.jit, donate_argnums=())
def _sc_call(xf, idx, pe):
    mesh = plsc.VectorSubcoreMesh(
        core_axis_name="c", subcore_axis_name="s", num_cores=NC, num_subcores=NS
    )
    return pl.kernel(
        _sc_body,
        out_type=jax.ShapeDtypeStruct((ROWS, D), jnp.float32),
        mesh=mesh,
        scratch_types=[
            pltpu.VMEM((RPW,), jnp.int32),
            pltpu.VMEM((CH, D), jnp.float32),
            pltpu.VMEM((CH, D), jnp.float32),
            pltpu.SemaphoreType.DMA,
        ],
    )(xf, idx, pe)


def kernel(x, rel_times, pe):
    xf = x.reshape(ROWS, D)
    idx = rel_times.reshape(ROWS).astype(jnp.int32)
    out = _sc_call(xf, idx, pe)
    return out.reshape(x.shape)


# speedup vs baseline: 2.3680x; 2.3680x over previous
import functools

import jax
import jax.numpy as jnp
from jax import lax
from jax.experimental import pallas as pl
from jax.experimental.pallas import tpu as pltpu
from jax.experimental.pallas import tpu_sc as plsc

NC, NS, L = 2, 16, 16
NW = NC * NS
D = 1024
ROWS = 4 * 2048
RPW = ROWS // NW
CH = 16
NCH = RPW // CH
VPR = D // L
NBUF = 3


def _sc_body(x_hbm, idx_hbm, pe_hbm, out_hbm, idx_v, *bufs_and_sems):
    x_v = bufs_and_sems[0:NBUF]
    pe_v = bufs_and_sems[NBUF : 2 * NBUF]
    xsem = bufs_and_sems[2 * NBUF : 3 * NBUF]
    gsem = bufs_and_sems[3 * NBUF : 4 * NBUF]
    osem = bufs_and_sems[4 * NBUF : 5 * NBUF]

    wid = lax.axis_index("s") * NC + lax.axis_index("c")
    base = wid * RPW
    pltpu.sync_copy(idx_hbm.at[pl.ds(base, RPW)], idx_v)

    xcp = [None] * NCH
    gcp = [None] * NCH
    ocp = [None] * NCH

    def issue_in(c):
        b = c % NBUF
        xcp[c] = pltpu.async_copy(
            x_hbm.at[pl.ds(base + c * CH, CH)], x_v[b], xsem[b]
        )
        gcp[c] = pltpu.async_copy(
            pe_hbm.at[idx_v.at[pl.ds(c * CH, CH)]], pe_v[b], gsem[b]
        )

    for c in range(min(NBUF, NCH)):
        issue_in(c)

    for c in range(NCH):
        b = c % NBUF
        if c > 0 and (c - 1 + NBUF) < NCH:
            ocp[c - 1].wait()
            issue_in(c - 1 + NBUF)
        xcp[c].wait()
        gcp[c].wait()

        xb = x_v[b]
        pb = pe_v[b]

        @plsc.parallel_loop(0, CH * VPR, 1, unroll=8)
        def _add(i):
            r = i // VPR
            j = (i % VPR) * L
            xb[r, pl.ds(j, L)] = xb[r, pl.ds(j, L)] + pb[r, pl.ds(j, L)]

        ocp[c] = pltpu.async_copy(
            x_v[b], out_hbm.at[pl.ds(base + c * CH, CH)], osem[b]
        )

    for c in range(max(0, NCH - NBUF), NCH):
        ocp[c].wait()


@jax.jit
def _sc_call(xf, idx, pe):
    mesh = plsc.VectorSubcoreMesh(
        core_axis_name="c", subcore_axis_name="s", num_cores=NC, num_subcores=NS
    )
    scratch = (
        [pltpu.VMEM((RPW,), jnp.int32)]
        + [pltpu.VMEM((CH, D), jnp.float32) for _ in range(2 * NBUF)]
        + [pltpu.SemaphoreType.DMA for _ in range(3 * NBUF)]
    )
    return pl.kernel(
        _sc_body,
        out_type=jax.ShapeDtypeStruct((ROWS, D), jnp.float32),
        mesh=mesh,
        scratch_types=scratch,
    )(xf, idx, pe)


def kernel(x, rel_times, pe):
    xf = x.reshape(ROWS, D)
    idx = rel_times.reshape(ROWS).astype(jnp.int32)
    out = _sc_call(xf, idx, pe)
    return out.reshape(x.shape)
